# Initial kernel scaffold; baseline (speedup 1.0000x reference)
import jax
import jax.numpy as jnp
from jax.experimental import pallas as pl


def kernel(logits, temperatures, top_ps, min_ps, presence_penalties, frequency_penalties, repetition_penalties, top_ks, output_tokens):
    raise NotImplementedError("write your pallas kernel here")


# speedup vs baseline: 5.2793x; 5.2793x over previous
import jax
import jax.numpy as jnp
from jax.experimental import pallas as pl
from jax.experimental.pallas import tpu as pltpu

_VPAD = 102400
_SUB = 8
_LANE = _VPAD // _SUB


def _row_kernel(lg_ref, scal_ref, tok_ref, probs_ref, nt_ref):
    i = pl.program_id(0)
    x0 = lg_ref[0]
    toks = tok_ref[0]

    rows = []
    for r in range(_SUB):
        gidx = jax.lax.broadcasted_iota(jnp.int32, (1, _LANE), 1) + r * _LANE
        cmp = (toks == gidx).astype(jnp.float32)
        rows.append(jnp.sum(cmp, axis=0, keepdims=True))
    counts = jnp.concatenate(rows, axis=0)

    temp = scal_ref[i, 0]
    top_p = scal_ref[i, 1]
    min_p = scal_ref[i, 2]
    pres = scal_ref[i, 3]
    freq = scal_ref[i, 4]
    repp = scal_ref[i, 5]
    kf = scal_ref[i, 6]

    mask = counts > 0.0
    rep = jnp.where(mask, repp, 1.0)
    x = jnp.where(x0 > 0.0, x0 / rep, x0 * rep)
    x = x - freq * counts
    x = x - pres * jnp.where(mask, 1.0, 0.0)
    x = x / temp

    m = jnp.max(x)
    e = jnp.exp(x - m)
    z = jnp.sum(e)
    p0 = e / z
    scaled = min_p * (1.0 / z)
    drop = p0 < scaled
    xm = jnp.where(drop, -jnp.inf, x)
    e2 = jnp.where(drop, 0.0, e)
    z2 = jnp.sum(e2)
    p = e2 / z2

    finite = xm > -jnp.inf
    finmin = jnp.min(jnp.where(finite, xm, m))
    lo0 = finmin - (jnp.abs(finmin) * 1e-5 + 1e-5)

    def body(_, st):
        lo_k, hi_k, lo_p, hi_p = st
        mid_k = lo_k * 0.5 + hi_k * 0.5
        c = jnp.sum(jnp.where(xm > mid_k, 1.0, 0.0))
        pk = c >= kf
        lo_k = jnp.where(pk, mid_k, lo_k)
        hi_k = jnp.where(pk, hi_k, mid_k)
        mid_p = lo_p * 0.5 + hi_p * 0.5
        msum = jnp.sum(jnp.where(xm > mid_p, p, 0.0))
        pp_ = msum > top_p
        lo_p = jnp.where(pp_, mid_p, lo_p)
        hi_p = jnp.where(pp_, hi_p, mid_p)
        return (lo_k, hi_k, lo_p, hi_p)

    lo_k, _, lo_p, _ = jax.lax.fori_loop(0, 50, body, (lo0, m, lo0, m))
    lo = jnp.maximum(lo_k, lo_p)
    keep = xm > lo

    e3 = jnp.where(keep, e2, 0.0)
    f = jnp.sum(e3)
    out = e3 / f
    probs_ref[0] = out

    mo = jnp.max(out)
    gidx_full = (jax.lax.broadcasted_iota(jnp.int32, (_SUB, _LANE), 0) * _LANE
                 + jax.lax.broadcasted_iota(jnp.int32, (_SUB, _LANE), 1))
    cand = jnp.where(out == mo, gidx_full, jnp.int32(2147483647))
    nt_ref[i] = jnp.min(cand)


def kernel(logits, temperatures, top_ps, min_ps, presence_penalties,
           frequency_penalties, repetition_penalties, top_ks, output_tokens):
    B, V = logits.shape
    L = output_tokens.shape[1]

    padded = jnp.full((B, _VPAD), -jnp.inf, logits.dtype)
    padded = padded.at[:, :V].set(logits)
    lg3 = padded.reshape(B, _SUB, _LANE)
    toks3 = output_tokens.astype(jnp.int32).reshape(B, L, 1)

    scal = jnp.stack(
        [temperatures, top_ps, min_ps, presence_penalties,
         frequency_penalties, repetition_penalties,
         top_ks.astype(jnp.float32)], axis=1).astype(jnp.float32)

    row_spec = pl.BlockSpec((1, _SUB, _LANE), lambda i: (i, 0, 0))
    smem_spec = pl.BlockSpec(memory_space=pltpu.SMEM)
    tok_spec = pl.BlockSpec((1, L, 1), lambda i: (i, 0, 0))

    probs3, nt = pl.pallas_call(
        _row_kernel,
        grid=(B,),
        in_specs=[row_spec, smem_spec, tok_spec],
        out_specs=[row_spec, smem_spec],
        out_shape=[
            jax.ShapeDtypeStruct((B, _SUB, _LANE), jnp.float32),
            jax.ShapeDtypeStruct((B,), jnp.int32),
        ],
    )(lg3, scal, toks3)

    final_probs = probs3.reshape(B, _VPAD)[:, :V]
    next_tokens = nt
    return final_probs, next_tokens
